# baseline (speedup 1.0000x reference)
import functools

import jax
import jax.numpy as jnp
from jax.experimental import pallas as pl
from jax.experimental.pallas import tpu as pltpu

_K = 3


def _routing_kernel(x_ref, W1_ref, b1_ref, W2_ref, b2_ref, we_hbm, be_hbm,
                    wpack_ref, vbe_ref, acc_ref, we_s, be_s, sem, bsem,
                    *, n_rows, n_experts, k_sel, d):
    i = pl.program_id(0)
    part = jnp.sum(x_ref[...], axis=0, keepdims=True)

    @pl.when(i == 0)
    def _():
        acc_ref[...] = part

    @pl.when(i > 0)
    def _():
        acc_ref[...] = acc_ref[...] + part

    @pl.when(i == pl.num_programs(0) - 1)
    def _():
        mean = acc_ref[...] * (1.0 / n_rows)
        h = jnp.maximum(
            jnp.dot(mean, W1_ref[...], preferred_element_type=jnp.float32)
            + b1_ref[...], 0.0)
        logits = (jnp.dot(h, W2_ref[...], preferred_element_type=jnp.float32)
                  + b2_ref[...])
        m = jnp.max(logits, axis=1, keepdims=True)
        ex = jnp.exp(logits - m)
        w = ex / jnp.sum(ex, axis=1, keepdims=True)
        lane = jax.lax.broadcasted_iota(jnp.int32, w.shape, 1)
        for k in range(k_sel):
            vk = jnp.max(w)
            ajs = jnp.min(jnp.where(w >= vk, lane, n_experts))
            pltpu.make_async_copy(we_hbm.at[ajs], we_s.at[k],
                                  sem.at[k]).start()
            pltpu.make_async_copy(be_hbm.at[ajs], be_s.at[k],
                                  bsem.at[k]).start()
            w = jnp.where(lane == ajs, -1.0, w)
            pltpu.make_async_copy(we_hbm.at[ajs], we_s.at[k],
                                  sem.at[k]).wait()
            pltpu.make_async_copy(be_hbm.at[ajs], be_s.at[k],
                                  bsem.at[k]).wait()
            wpack_ref[:, k * d:(k + 1) * d] = (
                we_s[k] * vk).astype(jnp.bfloat16)
            vbe_ref[:, k * d:(k + 1) * d] = be_s[k] * vk


def _expert_kernel(x_ref, wpack_ref, vbe_ref, out_ref, *, k_sel, d):
    xt = x_ref[...].astype(jnp.bfloat16)
    acc = None
    for k in range(k_sel):
        y = jnp.dot(xt, wpack_ref[:, k * d:(k + 1) * d],
                    preferred_element_type=jnp.float32)
        y = jnp.maximum(y + vbe_ref[:, k * d:(k + 1) * d], 0.0)
        acc = y if acc is None else acc + y
    out_ref[...] = acc


def kernel(x, W1, b1, W2, b2, We, be, top_k):
    del top_k
    n, d = x.shape
    h_dim = W1.shape[1]
    e_dim = W2.shape[1]
    k_sel = _K

    tile_a = 1024
    grid_a = n // tile_a
    wpack, vbe = pl.pallas_call(
        functools.partial(_routing_kernel, n_rows=n, n_experts=e_dim,
                          k_sel=k_sel, d=d),
        grid=(grid_a,),
        in_specs=[
            pl.BlockSpec((tile_a, d), lambda i: (i, 0)),
            pl.BlockSpec((d, h_dim), lambda i: (0, 0)),
            pl.BlockSpec((1, h_dim), lambda i: (0, 0)),
            pl.BlockSpec((h_dim, e_dim), lambda i: (0, 0)),
            pl.BlockSpec((1, e_dim), lambda i: (0, 0)),
            pl.BlockSpec(memory_space=pltpu.HBM),
            pl.BlockSpec(memory_space=pltpu.HBM),
        ],
        out_specs=[
            pl.BlockSpec((d, k_sel * d), lambda i: (0, 0)),
            pl.BlockSpec((1, k_sel * d), lambda i: (0, 0)),
        ],
        out_shape=[
            jax.ShapeDtypeStruct((d, k_sel * d), jnp.bfloat16),
            jax.ShapeDtypeStruct((1, k_sel * d), jnp.float32),
        ],
        scratch_shapes=[
            pltpu.VMEM((1, d), jnp.float32),
            pltpu.VMEM((k_sel, d, d), jnp.float32),
            pltpu.VMEM((k_sel, 1, d), jnp.float32),
            pltpu.SemaphoreType.DMA((k_sel,)),
            pltpu.SemaphoreType.DMA((k_sel,)),
        ],
        compiler_params=pltpu.CompilerParams(
            dimension_semantics=("arbitrary",)),
    )(x, W1, b1.reshape(1, h_dim), W2, b2.reshape(1, e_dim), We,
      be.reshape(e_dim, 1, d))

    tile_b = 1024
    grid_b = n // tile_b
    out = pl.pallas_call(
        functools.partial(_expert_kernel, k_sel=k_sel, d=d),
        grid=(grid_b,),
        in_specs=[
            pl.BlockSpec((tile_b, d), lambda i: (i, 0)),
            pl.BlockSpec((d, k_sel * d), lambda i: (0, 0)),
            pl.BlockSpec((1, k_sel * d), lambda i: (0, 0)),
        ],
        out_specs=pl.BlockSpec((tile_b, d), lambda i: (i, 0)),
        out_shape=jax.ShapeDtypeStruct((n, d), jnp.float32),
        compiler_params=pltpu.CompilerParams(
            dimension_semantics=("parallel",)),
    )(x, wpack, vbe)
    return out


# speedup vs baseline: 1.1317x; 1.1317x over previous
import functools

import jax
import jax.numpy as jnp
from jax.experimental import pallas as pl
from jax.experimental.pallas import tpu as pltpu

_K = 3


def _fused_kernel(x_ref, W1_ref, b1_ref, W2_ref, b2_ref, we_hbm, be_hbm,
                  out_ref, acc_ref, we_s, be_s, sem, bsem,
                  *, n_rows, n_experts, k_sel, d, col_t, g_steps):
    i = pl.program_id(0)

    @pl.when(i < g_steps)
    def _():
        part = jnp.sum(x_ref[...], axis=0, keepdims=True)

        @pl.when(i == 0)
        def _():
            acc_ref[...] = part

        @pl.when(i > 0)
        def _():
            acc_ref[...] = acc_ref[...] + part

    @pl.when(i == g_steps - 1)
    def _():
        mean = acc_ref[...] * (1.0 / n_rows)
        h = jnp.maximum(
            jnp.dot(mean, W1_ref[...], preferred_element_type=jnp.float32)
            + b1_ref[...], 0.0)
        logits = (jnp.dot(h, W2_ref[...], preferred_element_type=jnp.float32)
                  + b2_ref[...])
        m = jnp.max(logits, axis=1, keepdims=True)
        ex = jnp.exp(logits - m)
        w = ex / jnp.sum(ex, axis=1, keepdims=True)
        lane = jax.lax.broadcasted_iota(jnp.int32, w.shape, 1)
        vks = []
        for k in range(k_sel):
            vk = jnp.max(w)
            ajs = jnp.min(jnp.where(w >= vk, lane, n_experts))
            pltpu.make_async_copy(we_hbm.at[ajs], we_s.at[k],
                                  sem.at[k]).start()
            pltpu.make_async_copy(be_hbm.at[ajs], be_s.at[k],
                                  bsem.at[k]).start()
            w = jnp.where(lane == ajs, -1.0, w)
            vks.append(vk)
        for k in range(k_sel):
            pltpu.make_async_copy(we_hbm.at[0], we_s.at[k], sem.at[k]).wait()
            pltpu.make_async_copy(be_hbm.at[0], be_s.at[k], bsem.at[k]).wait()
            we_s[k] = we_s[k] * vks[k]
            be_s[k] = be_s[k] * vks[k]

    @pl.when(i >= g_steps)
    def _():
        xt = x_ref[...]
        for c in range(d // col_t):
            cs = c * col_t
            acc = None
            for k in range(k_sel):
                y = jnp.dot(xt, we_s[k, :, cs:cs + col_t],
                            preferred_element_type=jnp.float32)
                y = jnp.maximum(y + be_s[k, :, cs:cs + col_t], 0.0)
                acc = y if acc is None else acc + y
            out_ref[:, cs:cs + col_t] = acc


def kernel(x, W1, b1, W2, b2, We, be, top_k):
    del top_k
    n, d = x.shape
    h_dim = W1.shape[1]
    e_dim = W2.shape[1]
    k_sel = _K

    tile = 1024
    g = n // tile
    out = pl.pallas_call(
        functools.partial(_fused_kernel, n_rows=n, n_experts=e_dim,
                          k_sel=k_sel, d=d, col_t=256, g_steps=g),
        grid=(2 * g,),
        in_specs=[
            pl.BlockSpec((tile, d), lambda i: (jnp.where(i < g, i, i - g), 0)),
            pl.BlockSpec((d, h_dim), lambda i: (0, 0)),
            pl.BlockSpec((1, h_dim), lambda i: (0, 0)),
            pl.BlockSpec((h_dim, e_dim), lambda i: (0, 0)),
            pl.BlockSpec((1, e_dim), lambda i: (0, 0)),
            pl.BlockSpec(memory_space=pltpu.HBM),
            pl.BlockSpec(memory_space=pltpu.HBM),
        ],
        out_specs=pl.BlockSpec(
            (tile, d), lambda i: (jnp.where(i < g, 0, i - g), 0)),
        out_shape=jax.ShapeDtypeStruct((n, d), jnp.float32),
        scratch_shapes=[
            pltpu.VMEM((1, d), jnp.float32),
            pltpu.VMEM((k_sel, d, d), jnp.float32),
            pltpu.VMEM((k_sel, 1, d), jnp.float32),
            pltpu.SemaphoreType.DMA((k_sel,)),
            pltpu.SemaphoreType.DMA((k_sel,)),
        ],
        compiler_params=pltpu.CompilerParams(
            dimension_semantics=("arbitrary",)),
    )(x, W1, b1.reshape(1, h_dim), W2, b2.reshape(1, e_dim), We,
      be.reshape(e_dim, 1, d))
    return out
